# baseline (speedup 1.0000x reference)
import functools

import jax
import jax.numpy as jnp
from jax import lax
from jax.experimental import pallas as pl
from jax.experimental.pallas import tpu as pltpu
from jax.experimental.pallas import tpu_sc as plsc

N = 100000
D = 128
NUM_CLASSES = 1000

B = 160
G = 80
NCHUNKS = N // B
NW = 32
L = 16


def _mask_label_sc(x_hbm, y_hbm, m_hbm, tab_hbm, out_hbm,
                   yv, mv, idxv, xv, rv, sem_x, sem_g):
    wid = lax.axis_index("s") * 2 + lax.axis_index("c")
    cnt = (NCHUNKS - 1 - wid) // NW + 1

    def chunk_body(i, _):
        c = wid + i * NW
        base = c * B
        cx = pltpu.async_copy(x_hbm.at[pl.ds(base, B)], xv, sem_x)
        pltpu.sync_copy(y_hbm.at[pl.ds(base, B)], yv)
        pltpu.sync_copy(m_hbm.at[pl.ds(base, B)], mv)
        for j in range(B // L):
            sl = pl.ds(j * L, L)
            ivec = jnp.where(mv[sl] != 0, yv[sl], jnp.int32(NUM_CLASSES))
            idxv[j * L // G, pl.ds(j * L % G, L)] = ivec
        g0 = pltpu.async_copy(tab_hbm.at[idxv.at[0]], rv.at[pl.ds(0, G)],
                              sem_g)
        g1 = pltpu.async_copy(tab_hbm.at[idxv.at[1]], rv.at[pl.ds(G, G)],
                              sem_g)
        cx.wait()
        g0.wait()
        g1.wait()

        def add_row(r, carry):
            for cc in range(D // L):
                sl = pl.ds(cc * L, L)
                xv[r, sl] = xv[r, sl] + rv[r, sl]
            return carry

        lax.fori_loop(0, B, add_row, 0)
        pltpu.sync_copy(xv, out_hbm.at[pl.ds(base, B)])
        return _

    lax.fori_loop(0, cnt, chunk_body, 0)


@jax.jit
def _run(x, y, m_i32, table):
    mesh = plsc.VectorSubcoreMesh(core_axis_name="c", subcore_axis_name="s")
    f = functools.partial(
        pl.kernel,
        out_type=jax.ShapeDtypeStruct((N, D), jnp.float32),
        mesh=mesh,
        scratch_types=[
            pltpu.VMEM((B,), jnp.int32),
            pltpu.VMEM((B,), jnp.int32),
            pltpu.VMEM((B // G, G), jnp.int32),
            pltpu.VMEM((B, D), jnp.float32),
            pltpu.VMEM((B, D), jnp.float32),
            pltpu.SemaphoreType.DMA,
            pltpu.SemaphoreType.DMA,
        ],
    )(_mask_label_sc)
    return f(x, y, m_i32, table)


def kernel(x, y, mask, emb_weight):
    m_i32 = mask.astype(jnp.int32)
    table = jnp.concatenate(
        [emb_weight, jnp.zeros((8, D), jnp.float32)], axis=0)
    return _run(x, y, m_i32, table)
